# Initial kernel scaffold; baseline (speedup 1.0000x reference)
import jax
import jax.numpy as jnp
from jax.experimental import pallas as pl


def kernel(X, v_idx, e_idx, params):
    raise NotImplementedError("write your pallas kernel here")


# speedup vs baseline: 5.8449x; 5.8449x over previous
import functools

import jax
import jax.numpy as jnp
from jax import lax
from jax.experimental import pallas as pl
from jax.experimental.pallas import tpu as pltpu
from jax.experimental.pallas import tpu_sc as plsc

_N = 10000
_IN = 128
_HID = 128
_LAT = 64
_NE = 10000
_NNZ = 320000
_F = 128

_NC = 2
_NS = 16
_NW = _NC * _NS
_PPW = _NNZ // _NW
_CHUNK = 200
_NIT = _PPW // _CHUNK
_CCHUNK = 400
_CNIT = _PPW // _CCHUNK
_SEGP = 10240
_TROWS = _SEGP // _NS
_ZC = 160

_ROWBLK = 1000
_SROWBLK = 1024


def _mesh():
    return plsc.VectorSubcoreMesh(core_axis_name="c", subcore_axis_name="s",
                                  num_cores=_NC, num_subcores=_NS)


@functools.lru_cache(maxsize=1)
def _make_counts():
    @functools.partial(
        pl.kernel,
        out_type=jax.ShapeDtypeStruct((2 * _NC * _SEGP,), jnp.float32),
        mesh=_mesh(),
        compiler_params=pltpu.CompilerParams(needs_layout_passes=False),
        scratch_types=[
            pltpu.VMEM_SHARED((2 * _NS * _SEGP,), jnp.float32),
            pltpu.VMEM((_SEGP,), jnp.float32),
            pltpu.VMEM((_SEGP,), jnp.float32),
            pltpu.VMEM((_CCHUNK,), jnp.int32),
            pltpu.VMEM((_CCHUNK,), jnp.int32),
            pltpu.VMEM((_NS * _TROWS,), jnp.float32),
            pltpu.VMEM((_TROWS,), jnp.float32),
        ],
    )
    def _counts(vi, ei, out, stage, he, hv, vbuf, ebuf, red, res):
        cid = lax.axis_index("c")
        sid = lax.axis_index("s")
        wid = sid * _NC + cid
        zero16 = jnp.zeros((16,), jnp.float32)
        ones16 = jnp.ones((16,), jnp.float32)

        def _z(i, c):
            he[pl.ds(i * 16, 16)] = zero16
            hv[pl.ds(i * 16, 16)] = zero16
            return c

        lax.fori_loop(0, _SEGP // 16, _z, 0)

        base = wid * _PPW

        def _body(it, c):
            off = pl.multiple_of(base + it * _CCHUNK, 8)
            pltpu.sync_copy(vi.at[pl.ds(off, _CCHUNK)], vbuf)
            pltpu.sync_copy(ei.at[pl.ds(off, _CCHUNK)], ebuf)

            def _inner(j, c2):
                ev = ebuf[pl.ds(j * 16, 16)]
                plsc.addupdate_scatter(he, [ev], ones16)
                vv = vbuf[pl.ds(j * 16, 16)]
                plsc.addupdate_scatter(hv, [vv], ones16)
                return c2

            lax.fori_loop(0, _CCHUNK // 16, _inner, 0)
            return c

        lax.fori_loop(0, _CNIT, _body, 0)

        pltpu.sync_copy(he, stage.at[pl.ds((0 * _NS + sid) * _SEGP, _SEGP)])
        pltpu.sync_copy(hv, stage.at[pl.ds((1 * _NS + sid) * _SEGP, _SEGP)])
        plsc.subcore_barrier()

        for d in range(2):
            for t in range(_NS):
                pltpu.sync_copy(
                    stage.at[pl.ds((d * _NS + t) * _SEGP + sid * _TROWS,
                                   _TROWS)],
                    red.at[pl.ds(t * _TROWS, _TROWS)])

            def _rsum(i, c):
                acc = red[pl.ds(i * 16, 16)]
                for t in range(1, _NS):
                    acc = acc + red[pl.ds(t * _TROWS + i * 16, 16)]
                res[pl.ds(i * 16, 16)] = acc
                return c

            lax.fori_loop(0, _TROWS // 16, _rsum, 0)
            pltpu.sync_copy(
                res, out.at[pl.ds((d * _NC + cid) * _SEGP + sid * _TROWS,
                                  _TROWS)])

    return _counts


def _counts_sc(vi, ei):
    flat = _make_counts()(vi, ei)
    return flat.reshape(2, _NC, _SEGP)


@functools.lru_cache(maxsize=1)
def _make_segsum():
    @functools.partial(
        pl.kernel,
        out_type=jax.ShapeDtypeStruct((_NC, _SEGP, _F), jnp.float32),
        mesh=_mesh(),
        scratch_types=[
            pltpu.VMEM_SHARED((_SEGP, _F), jnp.float32),
            pltpu.VMEM((_CHUNK,), jnp.int32),
            pltpu.VMEM((_CHUNK,), jnp.int32),
            pltpu.VMEM((_CHUNK, _F), jnp.float32),
            pltpu.SemaphoreType.DMA,
        ],
    )
    def _segsum(table, gidx, sidx, out, accum, gv, sv, rows, sem):
        cid = lax.axis_index("c")
        sid = lax.axis_index("s")
        wid = sid * _NC + cid

        zero16 = jnp.zeros((16,), jnp.float32)

        def _zrow(r, carry):
            for j in range(_F // 16):
                rows[r, pl.ds(j * 16, 16)] = zero16
            return carry

        lax.fori_loop(0, _ZC, _zrow, 0)
        row0 = sid * _TROWS
        for k in range(_TROWS // _ZC):
            pltpu.sync_copy(rows.at[pl.ds(0, _ZC)],
                            accum.at[pl.ds(row0 + k * _ZC, _ZC)])
        plsc.subcore_barrier()

        base = wid * _PPW

        def _body(it, carry):
            off = pl.multiple_of(base + it * _CHUNK, 8)
            pltpu.sync_copy(gidx.at[pl.ds(off, _CHUNK)], gv)
            pltpu.sync_copy(sidx.at[pl.ds(off, _CHUNK)], sv)
            pltpu.async_copy(table.at[gv], rows, sem).wait()
            pltpu.sync_copy(rows, accum.at[sv], add=True)
            return carry

        lax.fori_loop(0, _NIT, _body, 0)
        plsc.subcore_barrier()

        for k in range(_TROWS // _ZC):
            sl = pl.ds(row0 + k * _ZC, _ZC)
            pltpu.sync_copy(accum.at[sl], rows.at[pl.ds(0, _ZC)])
            pltpu.sync_copy(rows.at[pl.ds(0, _ZC)], out.at[cid, sl])

    return _segsum


def _segsum_sc(table, gidx, sidx):
    return _make_segsum()(table, gidx, sidx)


def _mm_kernel(x_ref, w_ref, b_ref, o_ref):
    o_ref[...] = (jnp.dot(x_ref[...], w_ref[...],
                          preferred_element_type=jnp.float32) + b_ref[...])


def _mm(x, w, b):
    return pl.pallas_call(
        _mm_kernel,
        grid=(_N // _ROWBLK,),
        in_specs=[
            pl.BlockSpec((_ROWBLK, _IN), lambda i: (i, 0)),
            pl.BlockSpec((_IN, _F), lambda i: (0, 0)),
            pl.BlockSpec((1, _F), lambda i: (0, 0)),
        ],
        out_specs=pl.BlockSpec((_ROWBLK, _F), lambda i: (i, 0)),
        out_shape=jax.ShapeDtypeStruct((_N, _F), jnp.float32),
    )(x, w, b)


def _combine_kernel(p_ref, c_ref, o_ref):
    s = p_ref[0] + p_ref[1]
    c = jnp.maximum(c_ref[0] + c_ref[1], 1.0)
    o_ref[...] = s / c


def _combine(partials, cnt):
    return pl.pallas_call(
        _combine_kernel,
        grid=(_SEGP // _SROWBLK,),
        in_specs=[
            pl.BlockSpec((_NC, _SROWBLK, _F), lambda i: (0, i, 0)),
            pl.BlockSpec((_NC, _SROWBLK, 1), lambda i: (0, i, 0)),
        ],
        out_specs=pl.BlockSpec((_SROWBLK, _F), lambda i: (i, 0)),
        out_shape=jax.ShapeDtypeStruct((_SEGP, _F), jnp.float32),
    )(partials, cnt)


def _combine_relu_mm_kernel(p_ref, c_ref, w_ref, b_ref, o_ref):
    s = p_ref[0] + p_ref[1]
    c = jnp.maximum(c_ref[0] + c_ref[1], 1.0)
    h = jnp.maximum(s / c, 0.0)
    o_ref[...] = (jnp.dot(h, w_ref[...],
                          preferred_element_type=jnp.float32) + b_ref[...])


def _combine_relu_mm(partials, cnt, w, b):
    return pl.pallas_call(
        _combine_relu_mm_kernel,
        grid=(_N // _ROWBLK,),
        in_specs=[
            pl.BlockSpec((_NC, _ROWBLK, _F), lambda i: (0, i, 0)),
            pl.BlockSpec((_NC, _ROWBLK, 1), lambda i: (0, i, 0)),
            pl.BlockSpec((_HID, _F), lambda i: (0, 0)),
            pl.BlockSpec((1, _F), lambda i: (0, 0)),
        ],
        out_specs=pl.BlockSpec((_ROWBLK, _F), lambda i: (i, 0)),
        out_shape=jax.ShapeDtypeStruct((_N, _F), jnp.float32),
    )(partials, cnt, w, b)


def _leaky(x):
    return jnp.where(x >= 0, x, 0.2 * x)


def _encode_kernel(p_ref, c_ref, eps_ref, wmu_ref, bmu_ref, wlv_ref, blv_ref,
                   g_ref, b_ref, mu_ref, lv_ref, z_ref):
    s = p_ref[0, :_N] + p_ref[1, :_N]
    c = jnp.maximum(c_ref[0, :_N] + c_ref[1, :_N], 1.0)
    h = s / c
    m = jnp.mean(h, axis=0, keepdims=True)
    v = jnp.mean((h - m) ** 2, axis=0, keepdims=True)
    h = g_ref[...] * (h - m) / jnp.sqrt(v + 1e-5) + b_ref[...]
    h = _leaky(h)
    mu = jnp.dot(h, wmu_ref[...], preferred_element_type=jnp.float32) + bmu_ref[...]
    lv = jnp.dot(h, wlv_ref[...], preferred_element_type=jnp.float32) + blv_ref[...]
    mu_ref[...] = mu
    lv_ref[...] = lv
    std = jnp.maximum(jnp.exp(0.5 * lv), 1e-4)
    z_ref[...] = mu + eps_ref[...] * std


def _encode(partials, cnt, eps, wmu, bmu, wlv, blv, g, b):
    return pl.pallas_call(
        _encode_kernel,
        out_shape=(
            jax.ShapeDtypeStruct((_N, _LAT), jnp.float32),
            jax.ShapeDtypeStruct((_N, _LAT), jnp.float32),
            jax.ShapeDtypeStruct((_N, _LAT), jnp.float32),
        ),
    )(partials, cnt, eps, wmu, bmu, wlv, blv, g, b)


def _dec_struct_pre_kernel(z_ref, w_ref, b_ref, g_ref, bb_ref, o_ref):
    s = jnp.dot(z_ref[...], w_ref[...], preferred_element_type=jnp.float32) + b_ref[...]
    m = jnp.mean(s, axis=0, keepdims=True)
    v = jnp.mean((s - m) ** 2, axis=0, keepdims=True)
    s = g_ref[...] * (s - m) / jnp.sqrt(v + 1e-5) + bb_ref[...]
    o_ref[...] = _leaky(s)


def _dec_struct_pre(z, w, b, g, bb):
    return pl.pallas_call(
        _dec_struct_pre_kernel,
        out_shape=jax.ShapeDtypeStruct((_N, _HID), jnp.float32),
    )(z, w, b, g, bb)


def _dec_feat_kernel(z_ref, w1_ref, b1_ref, w2_ref, b2_ref, o_ref):
    f = jnp.dot(z_ref[...], w1_ref[...], preferred_element_type=jnp.float32) + b1_ref[...]
    f = _leaky(f)
    o_ref[...] = (jnp.dot(f, w2_ref[...],
                          preferred_element_type=jnp.float32) + b2_ref[...])


def _dec_feat(z, w1, b1, w2, b2):
    return pl.pallas_call(
        _dec_feat_kernel,
        out_shape=jax.ShapeDtypeStruct((_N, _IN), jnp.float32),
    )(z, w1, b1, w2, b2)


_COLBLK = 2048


def _struct_mm_kernel(s_ref, w_ref, b_ref, o_ref):
    x = jnp.dot(s_ref[...], w_ref[...], preferred_element_type=jnp.float32) + b_ref[...]
    o_ref[...] = jax.nn.sigmoid(x)


def _struct_mm(s_act, w, b):
    ncol = pl.cdiv(_N, _COLBLK)
    return pl.pallas_call(
        _struct_mm_kernel,
        grid=(_N // _ROWBLK, ncol),
        in_specs=[
            pl.BlockSpec((_ROWBLK, _HID), lambda i, j: (i, 0)),
            pl.BlockSpec((_HID, _COLBLK), lambda i, j: (0, j)),
            pl.BlockSpec((1, _COLBLK), lambda i, j: (0, j)),
        ],
        out_specs=pl.BlockSpec((_ROWBLK, _COLBLK), lambda i, j: (i, j)),
        out_shape=jax.ShapeDtypeStruct((_N, _N), jnp.float32),
    )(s_act, w, b)


def kernel(X, v_idx, e_idx, params):
    p = params
    vi = v_idx.astype(jnp.int32)
    ei = e_idx.astype(jnp.int32)

    counts = _counts_sc(vi, ei)
    cnt_e = counts[0].reshape(_NC, _SEGP, 1)
    cnt_v = counts[1].reshape(_NC, _SEGP, 1)

    h1 = _mm(X, p['W1'], p['b1'].reshape(1, _F))
    pe = _segsum_sc(h1, vi, ei)
    xe = _combine(pe, cnt_e)
    pv = _segsum_sc(xe, ei, vi)
    h2 = _combine_relu_mm(pv, cnt_v, p['W2'], p['b2'].reshape(1, _F))
    pe2 = _segsum_sc(h2, vi, ei)
    xe2 = _combine(pe2, cnt_e)
    pv2 = _segsum_sc(xe2, ei, vi)

    eps = jax.random.normal(jax.random.key(42), (_N, _LAT), jnp.float32)
    wmu = p['W_enc'][:, :_LAT]
    wlv = p['W_enc'][:, _LAT:]
    bmu = p['b_enc'][:_LAT].reshape(1, _LAT)
    blv = p['b_enc'][_LAT:].reshape(1, _LAT)
    g1 = p['bn1_g'].reshape(1, _HID)
    bb1 = p['bn1_b'].reshape(1, _HID)
    mu, logvar, z = _encode(pv2, cnt_v, eps, wmu, bmu, wlv, blv, g1, bb1)

    s_act = _dec_struct_pre(z, p['W_ds1'], p['b_ds1'].reshape(1, _HID),
                            p['bn2_g'].reshape(1, _HID),
                            p['bn2_b'].reshape(1, _HID))
    struct_recon = _struct_mm(s_act, p['W_ds2'], p['b_ds2'].reshape(1, _N))

    feat_recon = _dec_feat(z, p['W_df1'], p['b_df1'].reshape(1, _HID),
                           p['W_df2'], p['b_df2'].reshape(1, _IN))

    return struct_recon, feat_recon, mu, logvar


# speedup vs baseline: 8.2013x; 1.4032x over previous
import functools

import jax
import jax.numpy as jnp
from jax import lax
from jax.experimental import pallas as pl
from jax.experimental.pallas import tpu as pltpu
from jax.experimental.pallas import tpu_sc as plsc

_N = 10000
_IN = 128
_HID = 128
_LAT = 64
_NE = 10000
_NNZ = 320000
_F = 128

_NC = 2
_NS = 16
_NW = _NC * _NS
_PPW = _NNZ // _NW
_CHUNK = 80
_NIT = _PPW // _CHUNK
_CCHUNK = 400
_CNIT = _PPW // _CCHUNK
_SEGP = 10240
_TROWS = _SEGP // _NS
_ZC = 80

_ROWBLK = 1000
_SROWBLK = 1024


def _mesh():
    return plsc.VectorSubcoreMesh(core_axis_name="c", subcore_axis_name="s",
                                  num_cores=_NC, num_subcores=_NS)


@functools.lru_cache(maxsize=1)
def _make_counts():
    @functools.partial(
        pl.kernel,
        out_type=jax.ShapeDtypeStruct((2 * _NC * _SEGP,), jnp.float32),
        mesh=_mesh(),
        compiler_params=pltpu.CompilerParams(needs_layout_passes=False),
        scratch_types=[
            pltpu.VMEM_SHARED((2 * _NS * _SEGP,), jnp.float32),
            pltpu.VMEM((_SEGP,), jnp.float32),
            pltpu.VMEM((_SEGP,), jnp.float32),
            pltpu.VMEM((_CCHUNK,), jnp.int32),
            pltpu.VMEM((_CCHUNK,), jnp.int32),
            pltpu.VMEM((_NS * _TROWS,), jnp.float32),
            pltpu.VMEM((_TROWS,), jnp.float32),
        ],
    )
    def _counts(vi, ei, out, stage, he, hv, vbuf, ebuf, red, res):
        cid = lax.axis_index("c")
        sid = lax.axis_index("s")
        wid = sid * _NC + cid
        zero16 = jnp.zeros((16,), jnp.float32)
        ones16 = jnp.ones((16,), jnp.float32)

        def _z(i, c):
            he[pl.ds(i * 16, 16)] = zero16
            hv[pl.ds(i * 16, 16)] = zero16
            return c

        lax.fori_loop(0, _SEGP // 16, _z, 0)

        base = wid * _PPW

        def _body(it, c):
            off = pl.multiple_of(base + it * _CCHUNK, 8)
            pltpu.sync_copy(vi.at[pl.ds(off, _CCHUNK)], vbuf)
            pltpu.sync_copy(ei.at[pl.ds(off, _CCHUNK)], ebuf)

            def _inner(j, c2):
                ev = ebuf[pl.ds(j * 16, 16)]
                plsc.addupdate_scatter(he, [ev], ones16)
                vv = vbuf[pl.ds(j * 16, 16)]
                plsc.addupdate_scatter(hv, [vv], ones16)
                return c2

            lax.fori_loop(0, _CCHUNK // 16, _inner, 0)
            return c

        lax.fori_loop(0, _CNIT, _body, 0)

        pltpu.sync_copy(he, stage.at[pl.ds((0 * _NS + sid) * _SEGP, _SEGP)])
        pltpu.sync_copy(hv, stage.at[pl.ds((1 * _NS + sid) * _SEGP, _SEGP)])
        plsc.subcore_barrier()

        for d in range(2):
            for t in range(_NS):
                pltpu.sync_copy(
                    stage.at[pl.ds((d * _NS + t) * _SEGP + sid * _TROWS,
                                   _TROWS)],
                    red.at[pl.ds(t * _TROWS, _TROWS)])

            def _rsum(i, c):
                acc = red[pl.ds(i * 16, 16)]
                for t in range(1, _NS):
                    acc = acc + red[pl.ds(t * _TROWS + i * 16, 16)]
                res[pl.ds(i * 16, 16)] = acc
                return c

            lax.fori_loop(0, _TROWS // 16, _rsum, 0)
            pltpu.sync_copy(
                res, out.at[pl.ds((d * _NC + cid) * _SEGP + sid * _TROWS,
                                  _TROWS)])

    return _counts


def _counts_sc(vi, ei):
    flat = _make_counts()(vi, ei)
    return flat.reshape(2, _NC, _SEGP)


@functools.lru_cache(maxsize=1)
def _make_segsum():
    @functools.partial(
        pl.kernel,
        out_type=jax.ShapeDtypeStruct((_NC, _SEGP, _F), jnp.float32),
        mesh=_mesh(),
        scratch_types=[
            pltpu.VMEM_SHARED((_SEGP, _F), jnp.float32),
            pltpu.VMEM((_PPW,), jnp.int32),
            pltpu.VMEM((_PPW,), jnp.int32),
            pltpu.VMEM((_CHUNK, _F), jnp.float32),
            pltpu.VMEM((_CHUNK, _F), jnp.float32),
            pltpu.SemaphoreType.DMA,
            pltpu.SemaphoreType.DMA,
        ],
    )
    def _segsum(table, gidx, sidx, out, accum, gvall, svall, rows0, rows1,
                sem0, sem1):
        cid = lax.axis_index("c")
        sid = lax.axis_index("s")
        wid = sid * _NC + cid
        rws = (rows0, rows1)
        sems = (sem0, sem1)

        base = pl.multiple_of(wid * _PPW, 8)
        pltpu.sync_copy(gidx.at[pl.ds(base, _PPW)], gvall)
        pltpu.sync_copy(sidx.at[pl.ds(base, _PPW)], svall)

        zero16 = jnp.zeros((16,), jnp.float32)

        def _zrow(r, carry):
            for j in range(_F // 16):
                rows0[r, pl.ds(j * 16, 16)] = zero16
            return carry

        lax.fori_loop(0, _ZC, _zrow, 0)
        row0 = sid * _TROWS
        for k in range(_TROWS // _ZC):
            pltpu.sync_copy(rows0.at[pl.ds(0, _ZC)],
                            accum.at[pl.ds(row0 + k * _ZC, _ZC)])
        plsc.subcore_barrier()

        def _gather(k, b):
            idx = gvall.at[pl.ds(k * _CHUNK, _CHUNK)]
            pltpu.async_copy(table.at[idx], rws[b], sems[b])

        def _scat(k, b):
            pltpu.make_async_copy(table.at[gvall.at[pl.ds(0, _CHUNK)]],
                                  rws[b], sems[b]).wait()
            idx = svall.at[pl.ds(k * _CHUNK, _CHUNK)]
            pltpu.sync_copy(rws[b], accum.at[idx], add=True)

        _gather(0, 0)

        def _outer(o, carry):
            for b in range(2):
                k = o * 2 + b

                @pl.when(k + 1 < _NIT)
                def _():
                    _gather(k + 1, 1 - b)

                _scat(k, b)
            return carry

        lax.fori_loop(0, _NIT // 2, _outer, 0)
        _scat(_NIT - 1, 0)
        plsc.subcore_barrier()

        for k in range(_TROWS // _ZC):
            sl = pl.ds(row0 + k * _ZC, _ZC)
            pltpu.sync_copy(accum.at[sl], rows0.at[pl.ds(0, _ZC)])
            pltpu.sync_copy(rows0.at[pl.ds(0, _ZC)], out.at[cid, sl])

    return _segsum


def _segsum_sc(table, gidx, sidx):
    return _make_segsum()(table, gidx, sidx)


def _mm_kernel(x_ref, w_ref, b_ref, o_ref):
    o_ref[...] = (jnp.dot(x_ref[...], w_ref[...],
                          preferred_element_type=jnp.float32) + b_ref[...])


def _mm(x, w, b):
    return pl.pallas_call(
        _mm_kernel,
        grid=(_N // _ROWBLK,),
        in_specs=[
            pl.BlockSpec((_ROWBLK, _IN), lambda i: (i, 0)),
            pl.BlockSpec((_IN, _F), lambda i: (0, 0)),
            pl.BlockSpec((1, _F), lambda i: (0, 0)),
        ],
        out_specs=pl.BlockSpec((_ROWBLK, _F), lambda i: (i, 0)),
        out_shape=jax.ShapeDtypeStruct((_N, _F), jnp.float32),
    )(x, w, b)


def _combine_kernel(p_ref, c_ref, o_ref):
    s = p_ref[0] + p_ref[1]
    c = jnp.maximum(c_ref[0] + c_ref[1], 1.0)
    o_ref[...] = s / c


def _combine(partials, cnt):
    return pl.pallas_call(
        _combine_kernel,
        grid=(_SEGP // _SROWBLK,),
        in_specs=[
            pl.BlockSpec((_NC, _SROWBLK, _F), lambda i: (0, i, 0)),
            pl.BlockSpec((_NC, _SROWBLK, 1), lambda i: (0, i, 0)),
        ],
        out_specs=pl.BlockSpec((_SROWBLK, _F), lambda i: (i, 0)),
        out_shape=jax.ShapeDtypeStruct((_SEGP, _F), jnp.float32),
    )(partials, cnt)


def _combine_relu_mm_kernel(p_ref, c_ref, w_ref, b_ref, o_ref):
    s = p_ref[0] + p_ref[1]
    c = jnp.maximum(c_ref[0] + c_ref[1], 1.0)
    h = jnp.maximum(s / c, 0.0)
    o_ref[...] = (jnp.dot(h, w_ref[...],
                          preferred_element_type=jnp.float32) + b_ref[...])


def _combine_relu_mm(partials, cnt, w, b):
    return pl.pallas_call(
        _combine_relu_mm_kernel,
        grid=(_N // _ROWBLK,),
        in_specs=[
            pl.BlockSpec((_NC, _ROWBLK, _F), lambda i: (0, i, 0)),
            pl.BlockSpec((_NC, _ROWBLK, 1), lambda i: (0, i, 0)),
            pl.BlockSpec((_HID, _F), lambda i: (0, 0)),
            pl.BlockSpec((1, _F), lambda i: (0, 0)),
        ],
        out_specs=pl.BlockSpec((_ROWBLK, _F), lambda i: (i, 0)),
        out_shape=jax.ShapeDtypeStruct((_N, _F), jnp.float32),
    )(partials, cnt, w, b)


def _leaky(x):
    return jnp.where(x >= 0, x, 0.2 * x)


def _encode_kernel(p_ref, c_ref, eps_ref, wmu_ref, bmu_ref, wlv_ref, blv_ref,
                   g_ref, b_ref, mu_ref, lv_ref, z_ref):
    s = p_ref[0, :_N] + p_ref[1, :_N]
    c = jnp.maximum(c_ref[0, :_N] + c_ref[1, :_N], 1.0)
    h = s / c
    m = jnp.mean(h, axis=0, keepdims=True)
    v = jnp.mean((h - m) ** 2, axis=0, keepdims=True)
    h = g_ref[...] * (h - m) / jnp.sqrt(v + 1e-5) + b_ref[...]
    h = _leaky(h)
    mu = jnp.dot(h, wmu_ref[...], preferred_element_type=jnp.float32) + bmu_ref[...]
    lv = jnp.dot(h, wlv_ref[...], preferred_element_type=jnp.float32) + blv_ref[...]
    mu_ref[...] = mu
    lv_ref[...] = lv
    std = jnp.maximum(jnp.exp(0.5 * lv), 1e-4)
    z_ref[...] = mu + eps_ref[...] * std


def _encode(partials, cnt, eps, wmu, bmu, wlv, blv, g, b):
    return pl.pallas_call(
        _encode_kernel,
        out_shape=(
            jax.ShapeDtypeStruct((_N, _LAT), jnp.float32),
            jax.ShapeDtypeStruct((_N, _LAT), jnp.float32),
            jax.ShapeDtypeStruct((_N, _LAT), jnp.float32),
        ),
    )(partials, cnt, eps, wmu, bmu, wlv, blv, g, b)


def _dec_struct_pre_kernel(z_ref, w_ref, b_ref, g_ref, bb_ref, o_ref):
    s = jnp.dot(z_ref[...], w_ref[...], preferred_element_type=jnp.float32) + b_ref[...]
    m = jnp.mean(s, axis=0, keepdims=True)
    v = jnp.mean((s - m) ** 2, axis=0, keepdims=True)
    s = g_ref[...] * (s - m) / jnp.sqrt(v + 1e-5) + bb_ref[...]
    o_ref[...] = _leaky(s)


def _dec_struct_pre(z, w, b, g, bb):
    return pl.pallas_call(
        _dec_struct_pre_kernel,
        out_shape=jax.ShapeDtypeStruct((_N, _HID), jnp.float32),
    )(z, w, b, g, bb)


def _dec_feat_kernel(z_ref, w1_ref, b1_ref, w2_ref, b2_ref, o_ref):
    f = jnp.dot(z_ref[...], w1_ref[...], preferred_element_type=jnp.float32) + b1_ref[...]
    f = _leaky(f)
    o_ref[...] = (jnp.dot(f, w2_ref[...],
                          preferred_element_type=jnp.float32) + b2_ref[...])


def _dec_feat(z, w1, b1, w2, b2):
    return pl.pallas_call(
        _dec_feat_kernel,
        out_shape=jax.ShapeDtypeStruct((_N, _IN), jnp.float32),
    )(z, w1, b1, w2, b2)


_COLBLK = 2048


def _struct_mm_kernel(s_ref, w_ref, b_ref, o_ref):
    x = jnp.dot(s_ref[...], w_ref[...], preferred_element_type=jnp.float32) + b_ref[...]
    o_ref[...] = jax.nn.sigmoid(x)


def _struct_mm(s_act, w, b):
    ncol = pl.cdiv(_N, _COLBLK)
    return pl.pallas_call(
        _struct_mm_kernel,
        grid=(_N // _ROWBLK, ncol),
        in_specs=[
            pl.BlockSpec((_ROWBLK, _HID), lambda i, j: (i, 0)),
            pl.BlockSpec((_HID, _COLBLK), lambda i, j: (0, j)),
            pl.BlockSpec((1, _COLBLK), lambda i, j: (0, j)),
        ],
        out_specs=pl.BlockSpec((_ROWBLK, _COLBLK), lambda i, j: (i, j)),
        out_shape=jax.ShapeDtypeStruct((_N, _N), jnp.float32),
    )(s_act, w, b)


def kernel(X, v_idx, e_idx, params):
    p = params
    vi = v_idx.astype(jnp.int32)
    ei = e_idx.astype(jnp.int32)

    counts = _counts_sc(vi, ei)
    cnt_e = counts[0].reshape(_NC, _SEGP, 1)
    cnt_v = counts[1].reshape(_NC, _SEGP, 1)

    h1 = _mm(X, p['W1'], p['b1'].reshape(1, _F))
    pe = _segsum_sc(h1, vi, ei)
    xe = _combine(pe, cnt_e)
    pv = _segsum_sc(xe, ei, vi)
    h2 = _combine_relu_mm(pv, cnt_v, p['W2'], p['b2'].reshape(1, _F))
    pe2 = _segsum_sc(h2, vi, ei)
    xe2 = _combine(pe2, cnt_e)
    pv2 = _segsum_sc(xe2, ei, vi)

    eps = jax.random.normal(jax.random.key(42), (_N, _LAT), jnp.float32)
    wmu = p['W_enc'][:, :_LAT]
    wlv = p['W_enc'][:, _LAT:]
    bmu = p['b_enc'][:_LAT].reshape(1, _LAT)
    blv = p['b_enc'][_LAT:].reshape(1, _LAT)
    g1 = p['bn1_g'].reshape(1, _HID)
    bb1 = p['bn1_b'].reshape(1, _HID)
    mu, logvar, z = _encode(pv2, cnt_v, eps, wmu, bmu, wlv, blv, g1, bb1)

    s_act = _dec_struct_pre(z, p['W_ds1'], p['b_ds1'].reshape(1, _HID),
                            p['bn2_g'].reshape(1, _HID),
                            p['bn2_b'].reshape(1, _HID))
    struct_recon = _struct_mm(s_act, p['W_ds2'], p['b_ds2'].reshape(1, _N))

    feat_recon = _dec_feat(z, p['W_df1'], p['b_df1'].reshape(1, _HID),
                           p['W_df2'], p['b_df2'].reshape(1, _IN))

    return struct_recon, feat_recon, mu, logvar
